# Initial kernel scaffold; baseline (speedup 1.0000x reference)
import jax
import jax.numpy as jnp
from jax.experimental import pallas as pl


def kernel(x, edge_index, W_lin0, b_lin0, W_agg0, b_agg0, W_lin1, b_lin1, W_agg1, b_agg1, Wp1, bp1, Wp2, bp2):
    raise NotImplementedError("write your pallas kernel here")


# speedup vs baseline: 4.4129x; 4.4129x over previous
import functools

import jax
import jax.numpy as jnp
from jax import lax
from jax.experimental import pallas as pl
from jax.experimental.pallas import tpu as pltpu
from jax.experimental.pallas import tpu_sc as plsc

N = 10000
E = 320000
D = 128
H = 128
O = 64

NC = 2
NS = 16
NW = NC * NS
EPW = E // NW
CH = 80
NCH = EPW // CH
NP = 10240
RPT = NP // NS
ZR = 32

@functools.cache
def _mesh():
    return plsc.VectorSubcoreMesh(core_axis_name="c", subcore_axis_name="s")


def _sc_counts(dst):
    @functools.partial(
        pl.kernel,
        out_type=jax.ShapeDtypeStruct((NC, NP, H), jnp.float32),
        mesh=_mesh(),
        scratch_types=[
            pltpu.VMEM((CH, H), jnp.float32),
            pltpu.VMEM((CH,), jnp.int32),
            pltpu.VMEM((ZR, H), jnp.float32),
            pltpu.VMEM_SHARED((NP, H), jnp.float32),
        ],
    )
    def k(dst_hbm, out_hbm, ones_v, idx_v, zbuf, acc):
        c = lax.axis_index("c")
        s = lax.axis_index("s")

        @pl.loop(0, CH)
        def _(r):
            @pl.loop(0, H // 16)
            def _(q):
                ones_v[r, pl.ds(q * 16, 16)] = jnp.ones((16,), jnp.float32)

        @pl.loop(0, ZR)
        def _(r):
            @pl.loop(0, H // 16)
            def _(q):
                zbuf[r, pl.ds(q * 16, 16)] = jnp.zeros((16,), jnp.float32)

        @pl.loop(0, RPT // ZR)
        def _(i):
            pltpu.sync_copy(zbuf, acc.at[pl.ds(s * RPT + i * ZR, ZR)])

        plsc.subcore_barrier()

        base_e = (c * NS + s) * EPW

        @pl.loop(0, NCH)
        def _(j):
            pltpu.sync_copy(dst_hbm.at[pl.ds(base_e + j * CH, CH)], idx_v)
            pltpu.sync_copy(ones_v, acc.at[idx_v], add=True)

        plsc.subcore_barrier()
        pltpu.sync_copy(acc.at[pl.ds(s * RPT, RPT)],
                        out_hbm.at[c, pl.ds(s * RPT, RPT)])

    return k(dst)


def _sc_aggregate(h, src, dst):
    @functools.partial(
        pl.kernel,
        out_type=jax.ShapeDtypeStruct((NC, NP, H), jnp.float32),
        mesh=_mesh(),
        scratch_types=[
            pltpu.VMEM((CH, H), jnp.float32),
            pltpu.VMEM((CH,), jnp.int32),
            pltpu.VMEM((CH,), jnp.int32),
            pltpu.VMEM((ZR, H), jnp.float32),
            pltpu.VMEM_SHARED((NP, H), jnp.float32),
        ],
    )
    def k(h_hbm, src_hbm, dst_hbm, out_hbm, rows_v, src_v, dst_v, zbuf, acc):
        c = lax.axis_index("c")
        s = lax.axis_index("s")

        @pl.loop(0, ZR)
        def _(r):
            @pl.loop(0, H // 16)
            def _(q):
                zbuf[r, pl.ds(q * 16, 16)] = jnp.zeros((16,), jnp.float32)

        @pl.loop(0, RPT // ZR)
        def _(i):
            pltpu.sync_copy(zbuf, acc.at[pl.ds(s * RPT + i * ZR, ZR)])

        plsc.subcore_barrier()

        base_e = (c * NS + s) * EPW

        @pl.loop(0, NCH)
        def _(j):
            off = base_e + j * CH
            pltpu.sync_copy(src_hbm.at[pl.ds(off, CH)], src_v)
            pltpu.sync_copy(dst_hbm.at[pl.ds(off, CH)], dst_v)
            pltpu.sync_copy(h_hbm.at[src_v], rows_v)
            pltpu.sync_copy(rows_v, acc.at[dst_v], add=True)

        plsc.subcore_barrier()
        pltpu.sync_copy(acc.at[pl.ds(s * RPT, RPT)],
                        out_hbm.at[c, pl.ds(s * RPT, RPT)])

    return k(h, src, dst)


BB = 1000


def _dot(a, b):
    return lax.dot_general(a, b, (((1,), (0,)), ((), ())),
                           precision=lax.Precision.HIGHEST,
                           preferred_element_type=jnp.float32)


def _tc_lin0(x, wT, b):
    def body(x_ref, w_ref, b_ref, o_ref):
        o_ref[...] = jnp.maximum(_dot(x_ref[...], w_ref[...]) + b_ref[...], 0.0)

    return pl.pallas_call(
        body,
        grid=(N // BB,),
        in_specs=[pl.BlockSpec((BB, D), lambda i: (i, 0)),
                  pl.BlockSpec((D, H), lambda i: (0, 0)),
                  pl.BlockSpec((1, H), lambda i: (0, 0))],
        out_specs=pl.BlockSpec((BB, H), lambda i: (i, 0)),
        out_shape=jax.ShapeDtypeStruct((N, H), jnp.float32),
    )(x, wT, b)


def _sage_update(x_blk, acc_ref, cnt_ref, waT, ba):
    cnt = cnt_ref[0, :, 0:1] + cnt_ref[1, :, 0:1]
    agg = (acc_ref[0] + acc_ref[1]) / jnp.maximum(cnt, 1.0)
    cat = jnp.concatenate([x_blk, agg], axis=1)
    o = jnp.maximum(_dot(cat, waT) + ba, 0.0)
    nrm = jnp.maximum(jnp.sqrt(jnp.sum(o * o, axis=1, keepdims=True)), 1e-12)
    return o / nrm


def _tc_update0(x, acc, cnt, waT, ba, wlT, bl):
    def body(x_ref, acc_ref, cnt_ref, wa_ref, ba_ref, wl_ref, bl_ref,
             h0_ref, h1_ref):
        h0 = _sage_update(x_ref[...], acc_ref, cnt_ref, wa_ref[...], ba_ref[...])
        h0_ref[...] = h0
        h1_ref[...] = jnp.maximum(_dot(h0, wl_ref[...]) + bl_ref[...], 0.0)

    return pl.pallas_call(
        body,
        grid=(N // BB,),
        in_specs=[pl.BlockSpec((BB, H), lambda i: (i, 0)),
                  pl.BlockSpec((NC, BB, H), lambda i: (0, i, 0)),
                  pl.BlockSpec((NC, BB, H), lambda i: (0, i, 0)),
                  pl.BlockSpec((D + H, H), lambda i: (0, 0)),
                  pl.BlockSpec((1, H), lambda i: (0, 0)),
                  pl.BlockSpec((H, H), lambda i: (0, 0)),
                  pl.BlockSpec((1, H), lambda i: (0, 0))],
        out_specs=[pl.BlockSpec((BB, H), lambda i: (i, 0)),
                   pl.BlockSpec((BB, H), lambda i: (i, 0))],
        out_shape=[jax.ShapeDtypeStruct((N, H), jnp.float32),
                   jax.ShapeDtypeStruct((N, H), jnp.float32)],
    )(x, acc, cnt, waT, ba, wlT, bl)


def _tc_final(h0, acc, cnt, waT, ba, wp1T, bp1, wp2T, bp2):
    def body(x_ref, acc_ref, cnt_ref, wa_ref, ba_ref, w1_ref, b1_ref,
             w2_ref, b2_ref, o_ref):
        h1 = _sage_update(x_ref[...], acc_ref, cnt_ref, wa_ref[...], ba_ref[...])
        p = _dot(h1, w1_ref[...]) + b1_ref[...]
        q = _dot(p, w2_ref[...]) + b2_ref[...]
        m = jnp.max(q, axis=1, keepdims=True)
        lse = m + jnp.log(jnp.sum(jnp.exp(q - m), axis=1, keepdims=True))
        o_ref[...] = q - lse

    return pl.pallas_call(
        body,
        grid=(N // BB,),
        in_specs=[pl.BlockSpec((BB, H), lambda i: (i, 0)),
                  pl.BlockSpec((NC, BB, H), lambda i: (0, i, 0)),
                  pl.BlockSpec((NC, BB, H), lambda i: (0, i, 0)),
                  pl.BlockSpec((D + H, H), lambda i: (0, 0)),
                  pl.BlockSpec((1, H), lambda i: (0, 0)),
                  pl.BlockSpec((H, H), lambda i: (0, 0)),
                  pl.BlockSpec((1, H), lambda i: (0, 0)),
                  pl.BlockSpec((H, O), lambda i: (0, 0)),
                  pl.BlockSpec((1, O), lambda i: (0, 0))],
        out_specs=pl.BlockSpec((BB, O), lambda i: (i, 0)),
        out_shape=jax.ShapeDtypeStruct((N, O), jnp.float32),
    )(h0, acc, cnt, waT, ba, wp1T, bp1, wp2T, bp2)


def kernel(x, edge_index, W_lin0, b_lin0, W_agg0, b_agg0,
           W_lin1, b_lin1, W_agg1, b_agg1, Wp1, bp1, Wp2, bp2):
    src = edge_index[0]
    dst = edge_index[1]
    cnt = _sc_counts(dst)
    h = _tc_lin0(x, W_lin0.T, b_lin0.reshape(1, H))
    acc0 = _sc_aggregate(h, src, dst)
    h0, h1l = _tc_update0(x, acc0, cnt, W_agg0.T, b_agg0.reshape(1, H),
                          W_lin1.T, b_lin1.reshape(1, H))
    acc1 = _sc_aggregate(h1l, src, dst)
    return _tc_final(h0, acc1, cnt, W_agg1.T, b_agg1.reshape(1, H),
                     Wp1.T, bp1.reshape(1, H), Wp2.T, bp2.reshape(1, O))


# speedup vs baseline: 9.0090x; 2.0415x over previous
import functools

import jax
import jax.numpy as jnp
from jax import lax
from jax.experimental import pallas as pl
from jax.experimental.pallas import tpu as pltpu
from jax.experimental.pallas import tpu_sc as plsc

N = 10000
E = 320000
D = 128
H = 128
O = 64

NC = 2
NS = 16
NW = NC * NS
CH = 128
NCH = 80
NPAIR = NCH // 2
EPW = CH * NCH
EP = NW * EPW
NP = 10240
RPT = NP // NS
ZB = 64

@functools.cache
def _mesh():
    return plsc.VectorSubcoreMesh(core_axis_name="c", subcore_axis_name="s")


def _sc_counts(dst_p):
    NB = 2

    @functools.partial(
        pl.kernel,
        out_type=jax.ShapeDtypeStruct((NC, NP, H), jnp.float32),
        mesh=_mesh(),
        scratch_types=[
            [pltpu.VMEM((CH,), jnp.int32) for _ in range(NB)],
            [pltpu.SemaphoreType.DMA for _ in range(NB)],
            [pltpu.SemaphoreType.DMA for _ in range(NB)],
            pltpu.VMEM((CH, H), jnp.float32),
            pltpu.VMEM((ZB, H), jnp.float32),
            pltpu.VMEM_SHARED((NP, H), jnp.float32),
        ],
    )
    def k(dst_hbm, out_hbm, dstb, isems, ssems, ones_v, zbuf, acc):
        c = lax.axis_index("c")
        s = lax.axis_index("s")
        base_e = (c * NS + s) * EPW

        def start_idx(j, b):
            pltpu.async_copy(dst_hbm.at[pl.ds(base_e + j * CH, CH)],
                             dstb[b], isems[b])

        def wait_idx(b):
            pltpu.make_async_copy(dst_hbm.at[pl.ds(base_e, CH)], dstb[b],
                                  isems[b]).wait()

        def start_scat(b):
            pltpu.async_copy(ones_v, acc.at[dstb[b]], ssems[b], add=True)

        def wait_scat(b):
            pltpu.make_async_copy(ones_v, acc.at[dstb[b]], ssems[b]).wait()

        for b in range(NB):
            start_idx(b, b)

        @pl.loop(0, CH)
        def _(r):
            @pl.loop(0, H // 16)
            def _(q):
                ones_v[r, pl.ds(q * 16, 16)] = jnp.ones((16,), jnp.float32)

        @pl.loop(0, ZB)
        def _(r):
            @pl.loop(0, H // 16)
            def _(q):
                zbuf[r, pl.ds(q * 16, 16)] = jnp.zeros((16,), jnp.float32)

        @pl.loop(0, RPT // ZB)
        def _(i):
            pltpu.sync_copy(zbuf, acc.at[pl.ds(s * RPT + i * ZB, ZB)])

        plsc.subcore_barrier()

        @pl.loop(0, NCH // NB)
        def _(i):
            for b in range(NB):
                wait_idx(b)
                start_scat(b)

            for b in range(NB):
                @pl.when(i < NCH // NB - 1)
                def _():
                    wait_scat(b)
                    start_idx(NB * i + NB + b, b)

        for b in range(NB):
            wait_scat(b)

        plsc.subcore_barrier()
        pltpu.sync_copy(acc.at[pl.ds(s * RPT, RPT)],
                        out_hbm.at[c, pl.ds(s * RPT, RPT)])

    return k(dst_p)


def _sc_aggregate(h, src3, dst3):
    @functools.partial(
        pl.kernel,
        out_type=jax.ShapeDtypeStruct((NC, NP, H), jnp.float32),
        mesh=_mesh(),
        scratch_types=[
            [pltpu.VMEM((CH,), jnp.int32) for _ in range(2)],
            [pltpu.VMEM((CH,), jnp.int32) for _ in range(2)],
            [pltpu.VMEM((CH, H), jnp.float32) for _ in range(2)],
            pltpu.VMEM((ZB, H), jnp.float32),
            pltpu.VMEM_SHARED((NP, H), jnp.float32),
            [pltpu.SemaphoreType.DMA for _ in range(2)],
            [pltpu.SemaphoreType.DMA for _ in range(2)],
            [pltpu.SemaphoreType.DMA for _ in range(2)],
            [pltpu.SemaphoreType.DMA for _ in range(2)],
        ],
    )
    def k(h_hbm, src_hbm, dst_hbm, out_hbm, srcb, dstb, rows,
          zbuf, acc, isem, dsem, gsem, ssem):
        c = lax.axis_index("c")
        s = lax.axis_index("s")
        base_e = (c * NS + s) * EPW

        def start_src(j, b):
            pltpu.async_copy(src_hbm.at[pl.ds(base_e + j * CH, CH)],
                             srcb[b], isem[b])

        def wait_src(b):
            pltpu.make_async_copy(src_hbm.at[pl.ds(base_e, CH)], srcb[b],
                                  isem[b]).wait()

        def start_dst(j, b):
            pltpu.async_copy(dst_hbm.at[pl.ds(base_e + j * CH, CH)],
                             dstb[b], dsem[b])

        def wait_dst(b):
            pltpu.make_async_copy(dst_hbm.at[pl.ds(base_e, CH)], dstb[b],
                                  dsem[b]).wait()

        start_src(0, 0)
        start_src(1, 1)

        @pl.loop(0, ZB)
        def _(r):
            @pl.loop(0, H // 16)
            def _(q):
                zbuf[r, pl.ds(q * 16, 16)] = jnp.zeros((16,), jnp.float32)

        @pl.loop(0, RPT // ZB)
        def _(i):
            pltpu.sync_copy(zbuf, acc.at[pl.ds(s * RPT + i * ZB, ZB)])

        plsc.subcore_barrier()

        def start_gather(b):
            pltpu.async_copy(h_hbm.at[srcb[b]], rows[b], gsem[b])

        def wait_gather(b):
            pltpu.make_async_copy(h_hbm.at[srcb[b]], rows[b], gsem[b]).wait()

        def start_scat(b):
            pltpu.async_copy(rows[b], acc.at[dstb[b]], ssem[b], add=True)

        def wait_scat(b):
            pltpu.make_async_copy(rows[b], acc.at[dstb[b]], ssem[b]).wait()

        @pl.loop(0, NPAIR)
        def _(i):
            wait_src(0)

            @pl.when(i > 0)
            def _():
                wait_scat(0)

            start_dst(2 * i, 0)
            start_gather(0)

            @pl.when(i > 0)
            def _():
                wait_gather(1)
                wait_dst(1)
                start_scat(1)
                start_src(2 * i + 1, 1)

            wait_src(1)

            @pl.when(i > 0)
            def _():
                wait_scat(1)

            start_dst(2 * i + 1, 1)
            start_gather(1)

            wait_gather(0)
            wait_dst(0)
            start_scat(0)

            @pl.when(i < NPAIR - 1)
            def _():
                start_src(2 * i + 2, 0)

        wait_gather(1)
        wait_dst(1)
        start_scat(1)
        wait_scat(0)
        wait_scat(1)
        plsc.subcore_barrier()
        pltpu.sync_copy(acc.at[pl.ds(s * RPT, RPT)],
                        out_hbm.at[c, pl.ds(s * RPT, RPT)])

    return k(h, src3, dst3)


BB = 1000


def _dot(a, b):
    return lax.dot_general(a, b, (((1,), (0,)), ((), ())),
                           precision=lax.Precision.HIGHEST,
                           preferred_element_type=jnp.float32)


def _tc_lin0(x, wT, b):
    def body(x_ref, w_ref, b_ref, o_ref):
        o_ref[...] = jnp.maximum(_dot(x_ref[...], w_ref[...]) + b_ref[...], 0.0)

    return pl.pallas_call(
        body,
        grid=(N // BB,),
        in_specs=[pl.BlockSpec((BB, D), lambda i: (i, 0)),
                  pl.BlockSpec((D, H), lambda i: (0, 0)),
                  pl.BlockSpec((1, H), lambda i: (0, 0))],
        out_specs=pl.BlockSpec((BB, H), lambda i: (i, 0)),
        out_shape=jax.ShapeDtypeStruct((N, H), jnp.float32),
    )(x, wT, b)


def _sage_update(x_blk, acc_ref, cnt_ref, waT, ba):
    cnt = cnt_ref[0, :, 0:1] + cnt_ref[1, :, 0:1]
    agg = (acc_ref[0] + acc_ref[1]) / jnp.maximum(cnt, 1.0)
    cat = jnp.concatenate([x_blk, agg], axis=1)
    o = jnp.maximum(_dot(cat, waT) + ba, 0.0)
    nrm = jnp.maximum(jnp.sqrt(jnp.sum(o * o, axis=1, keepdims=True)), 1e-12)
    return o / nrm


def _tc_update0(x, acc, cnt, waT, ba, wlT, bl):
    def body(x_ref, acc_ref, cnt_ref, wa_ref, ba_ref, wl_ref, bl_ref,
             h0_ref, h1_ref):
        h0 = _sage_update(x_ref[...], acc_ref, cnt_ref, wa_ref[...], ba_ref[...])
        h0_ref[...] = h0
        h1_ref[...] = jnp.maximum(_dot(h0, wl_ref[...]) + bl_ref[...], 0.0)

    return pl.pallas_call(
        body,
        grid=(N // BB,),
        in_specs=[pl.BlockSpec((BB, H), lambda i: (i, 0)),
                  pl.BlockSpec((NC, BB, H), lambda i: (0, i, 0)),
                  pl.BlockSpec((NC, BB, H), lambda i: (0, i, 0)),
                  pl.BlockSpec((D + H, H), lambda i: (0, 0)),
                  pl.BlockSpec((1, H), lambda i: (0, 0)),
                  pl.BlockSpec((H, H), lambda i: (0, 0)),
                  pl.BlockSpec((1, H), lambda i: (0, 0))],
        out_specs=[pl.BlockSpec((BB, H), lambda i: (i, 0)),
                   pl.BlockSpec((BB, H), lambda i: (i, 0))],
        out_shape=[jax.ShapeDtypeStruct((N, H), jnp.float32),
                   jax.ShapeDtypeStruct((N, H), jnp.float32)],
    )(x, acc, cnt, waT, ba, wlT, bl)


def _tc_final(h0, acc, cnt, waT, ba, wp1T, bp1, wp2T, bp2):
    def body(x_ref, acc_ref, cnt_ref, wa_ref, ba_ref, w1_ref, b1_ref,
             w2_ref, b2_ref, o_ref):
        h1 = _sage_update(x_ref[...], acc_ref, cnt_ref, wa_ref[...], ba_ref[...])
        p = _dot(h1, w1_ref[...]) + b1_ref[...]
        q = _dot(p, w2_ref[...]) + b2_ref[...]
        m = jnp.max(q, axis=1, keepdims=True)
        lse = m + jnp.log(jnp.sum(jnp.exp(q - m), axis=1, keepdims=True))
        o_ref[...] = q - lse

    return pl.pallas_call(
        body,
        grid=(N // BB,),
        in_specs=[pl.BlockSpec((BB, H), lambda i: (i, 0)),
                  pl.BlockSpec((NC, BB, H), lambda i: (0, i, 0)),
                  pl.BlockSpec((NC, BB, H), lambda i: (0, i, 0)),
                  pl.BlockSpec((D + H, H), lambda i: (0, 0)),
                  pl.BlockSpec((1, H), lambda i: (0, 0)),
                  pl.BlockSpec((H, H), lambda i: (0, 0)),
                  pl.BlockSpec((1, H), lambda i: (0, 0)),
                  pl.BlockSpec((H, O), lambda i: (0, 0)),
                  pl.BlockSpec((1, O), lambda i: (0, 0))],
        out_specs=pl.BlockSpec((BB, O), lambda i: (i, 0)),
        out_shape=jax.ShapeDtypeStruct((N, O), jnp.float32),
    )(h0, acc, cnt, waT, ba, wp1T, bp1, wp2T, bp2)


def kernel(x, edge_index, W_lin0, b_lin0, W_agg0, b_agg0,
           W_lin1, b_lin1, W_agg1, b_agg1, Wp1, bp1, Wp2, bp2):
    npad = EPW - E // NW
    pad_src = (jnp.arange(npad, dtype=jnp.int32)[None, :] * 41
               + 97 * jnp.arange(NW, dtype=jnp.int32)[:, None]) % N
    pad_dst = jnp.broadcast_to(N + jnp.arange(npad, dtype=jnp.int32)[None, :],
                               (NW, npad))
    src_p = jnp.concatenate(
        [edge_index[0].reshape(NW, -1), pad_src], axis=1).reshape(EP)
    dst_p = jnp.concatenate(
        [edge_index[1].reshape(NW, -1), pad_dst], axis=1).reshape(EP)
    cnt = _sc_counts(dst_p)
    h = _tc_lin0(x, W_lin0.T, b_lin0.reshape(1, H))
    acc0 = _sc_aggregate(h, src_p, dst_p)
    h0, h1l = _tc_update0(x, acc0, cnt, W_agg0.T, b_agg0.reshape(1, H),
                          W_lin1.T, b_lin1.reshape(1, H))
    acc1 = _sc_aggregate(h1l, src_p, dst_p)
    return _tc_final(h0, acc1, cnt, W_agg1.T, b_agg1.reshape(1, H),
                     Wp1.T, bp1.reshape(1, H), Wp2.T, bp2.reshape(1, O))


# speedup vs baseline: 9.0926x; 1.0093x over previous
import functools

import jax
import jax.numpy as jnp
from jax import lax
from jax.experimental import pallas as pl
from jax.experimental.pallas import tpu as pltpu
from jax.experimental.pallas import tpu_sc as plsc

N = 10000
E = 320000
D = 128
H = 128
O = 64

NC = 2
NS = 16
NW = NC * NS
CH = 128
NCH = 80
NPAIR = NCH // 2
EPW = CH * NCH
EP = NW * EPW
NP = 10240
RPT = NP // NS
ZB = 64

@functools.cache
def _mesh():
    return plsc.VectorSubcoreMesh(core_axis_name="c", subcore_axis_name="s")


def _sc_aggregate(h, src3, dst3, with_counts=False):
    out_t = jax.ShapeDtypeStruct((NC, NP, H), jnp.float32)

    @functools.partial(
        pl.kernel,
        out_type=[out_t, out_t] if with_counts else out_t,
        mesh=_mesh(),
        scratch_types=[
            [pltpu.VMEM((CH,), jnp.int32) for _ in range(2)],
            [pltpu.VMEM((CH,), jnp.int32) for _ in range(2)],
            [pltpu.VMEM((CH, H), jnp.float32) for _ in range(2)],
            pltpu.VMEM((ZB, H), jnp.float32),
            pltpu.VMEM_SHARED((NP, H), jnp.float32),
            [pltpu.SemaphoreType.DMA for _ in range(2)],
            [pltpu.SemaphoreType.DMA for _ in range(2)],
            [pltpu.SemaphoreType.DMA for _ in range(2)],
            [pltpu.SemaphoreType.DMA for _ in range(2)],
        ],
    )
    def k(h_hbm, src_hbm, dst_hbm, *rest):
        if with_counts:
            cnt_hbm, out_hbm = rest[0], rest[1]
            srcb, dstb, rows, zbuf, acc, isem, dsem, gsem, ssem = rest[2:]
        else:
            out_hbm = rest[0]
            srcb, dstb, rows, zbuf, acc, isem, dsem, gsem, ssem = rest[1:]
        c = lax.axis_index("c")
        s = lax.axis_index("s")
        base_e = (c * NS + s) * EPW

        def start_src(j, b):
            pltpu.async_copy(src_hbm.at[pl.ds(base_e + j * CH, CH)],
                             srcb[b], isem[b])

        def wait_src(b):
            pltpu.make_async_copy(src_hbm.at[pl.ds(base_e, CH)], srcb[b],
                                  isem[b]).wait()

        def start_dst(j, b):
            pltpu.async_copy(dst_hbm.at[pl.ds(base_e + j * CH, CH)],
                             dstb[b], dsem[b])

        def wait_dst(b):
            pltpu.make_async_copy(dst_hbm.at[pl.ds(base_e, CH)], dstb[b],
                                  dsem[b]).wait()

        start_src(0, 0)
        start_src(1, 1)

        @pl.loop(0, ZB)
        def _(r):
            @pl.loop(0, H // 16)
            def _(q):
                zbuf[r, pl.ds(q * 16, 16)] = jnp.zeros((16,), jnp.float32)

        @pl.loop(0, RPT // ZB)
        def _(i):
            pltpu.sync_copy(zbuf, acc.at[pl.ds(s * RPT + i * ZB, ZB)])

        plsc.subcore_barrier()

        if with_counts:
            @pl.loop(0, CH)
            def _(r):
                @pl.loop(0, H // 16)
                def _(q):
                    rows[1][r, pl.ds(q * 16, 16)] = jnp.ones((16,),
                                                             jnp.float32)

            def wait_ones_scat(b):
                pltpu.make_async_copy(rows[1], acc.at[dstb[b]],
                                      ssem[b]).wait()

            start_dst(0, 0)
            start_dst(1, 1)

            @pl.loop(0, NPAIR)
            def _(i):
                for b in range(2):
                    wait_dst(b)
                    pltpu.async_copy(rows[1], acc.at[dstb[b]], ssem[b],
                                     add=True)

                for b in range(2):
                    @pl.when(i < NPAIR - 1)
                    def _():
                        wait_ones_scat(b)
                        start_dst(2 * i + 2 + b, b)

            for b in range(2):
                wait_ones_scat(b)

            plsc.subcore_barrier()
            pltpu.sync_copy(acc.at[pl.ds(s * RPT, RPT)],
                            cnt_hbm.at[c, pl.ds(s * RPT, RPT)])

            @pl.loop(0, RPT // ZB)
            def _(i):
                pltpu.sync_copy(zbuf, acc.at[pl.ds(s * RPT + i * ZB, ZB)])

            plsc.subcore_barrier()

        def start_gather(b):
            pltpu.async_copy(h_hbm.at[srcb[b]], rows[b], gsem[b])

        def wait_gather(b):
            pltpu.make_async_copy(h_hbm.at[srcb[b]], rows[b], gsem[b]).wait()

        def start_scat(b):
            pltpu.async_copy(rows[b], acc.at[dstb[b]], ssem[b], add=True)

        def wait_scat(b):
            pltpu.make_async_copy(rows[b], acc.at[dstb[b]], ssem[b]).wait()

        @pl.loop(0, NPAIR)
        def _(i):
            wait_src(0)

            @pl.when(i > 0)
            def _():
                wait_scat(0)

            start_dst(2 * i, 0)
            start_gather(0)

            @pl.when(i > 0)
            def _():
                wait_gather(1)
                wait_dst(1)
                start_scat(1)
                start_src(2 * i + 1, 1)

            wait_src(1)

            @pl.when(i > 0)
            def _():
                wait_scat(1)

            start_dst(2 * i + 1, 1)
            start_gather(1)

            wait_gather(0)
            wait_dst(0)
            start_scat(0)

            @pl.when(i < NPAIR - 1)
            def _():
                start_src(2 * i + 2, 0)

        wait_gather(1)
        wait_dst(1)
        start_scat(1)
        wait_scat(0)
        wait_scat(1)
        plsc.subcore_barrier()
        pltpu.sync_copy(acc.at[pl.ds(s * RPT, RPT)],
                        out_hbm.at[c, pl.ds(s * RPT, RPT)])

    return k(h, src3, dst3)


BB = 1000


def _dot(a, b):
    return lax.dot_general(a, b, (((1,), (0,)), ((), ())),
                           precision=lax.Precision.HIGHEST,
                           preferred_element_type=jnp.float32)


def _tc_lin0(x, wT, b):
    def body(x_ref, w_ref, b_ref, o_ref):
        o_ref[...] = jnp.maximum(_dot(x_ref[...], w_ref[...]) + b_ref[...], 0.0)

    return pl.pallas_call(
        body,
        grid=(N // BB,),
        in_specs=[pl.BlockSpec((BB, D), lambda i: (i, 0)),
                  pl.BlockSpec((D, H), lambda i: (0, 0)),
                  pl.BlockSpec((1, H), lambda i: (0, 0))],
        out_specs=pl.BlockSpec((BB, H), lambda i: (i, 0)),
        out_shape=jax.ShapeDtypeStruct((N, H), jnp.float32),
    )(x, wT, b)


def _sage_update(x_blk, acc_ref, cnt_ref, waT, ba):
    cnt = cnt_ref[0, :, 0:1] + cnt_ref[1, :, 0:1]
    agg = (acc_ref[0] + acc_ref[1]) / jnp.maximum(cnt, 1.0)
    cat = jnp.concatenate([x_blk, agg], axis=1)
    o = jnp.maximum(_dot(cat, waT) + ba, 0.0)
    nrm = jnp.maximum(jnp.sqrt(jnp.sum(o * o, axis=1, keepdims=True)), 1e-12)
    return o / nrm


def _tc_update0(x, acc, cnt, waT, ba, wlT, bl):
    def body(x_ref, acc_ref, cnt_ref, wa_ref, ba_ref, wl_ref, bl_ref,
             h0_ref, h1_ref):
        h0 = _sage_update(x_ref[...], acc_ref, cnt_ref, wa_ref[...], ba_ref[...])
        h0_ref[...] = h0
        h1_ref[...] = jnp.maximum(_dot(h0, wl_ref[...]) + bl_ref[...], 0.0)

    return pl.pallas_call(
        body,
        grid=(N // BB,),
        in_specs=[pl.BlockSpec((BB, H), lambda i: (i, 0)),
                  pl.BlockSpec((NC, BB, H), lambda i: (0, i, 0)),
                  pl.BlockSpec((NC, BB, H), lambda i: (0, i, 0)),
                  pl.BlockSpec((D + H, H), lambda i: (0, 0)),
                  pl.BlockSpec((1, H), lambda i: (0, 0)),
                  pl.BlockSpec((H, H), lambda i: (0, 0)),
                  pl.BlockSpec((1, H), lambda i: (0, 0))],
        out_specs=[pl.BlockSpec((BB, H), lambda i: (i, 0)),
                   pl.BlockSpec((BB, H), lambda i: (i, 0))],
        out_shape=[jax.ShapeDtypeStruct((N, H), jnp.float32),
                   jax.ShapeDtypeStruct((N, H), jnp.float32)],
    )(x, acc, cnt, waT, ba, wlT, bl)


def _tc_final(h0, acc, cnt, waT, ba, wp1T, bp1, wp2T, bp2):
    def body(x_ref, acc_ref, cnt_ref, wa_ref, ba_ref, w1_ref, b1_ref,
             w2_ref, b2_ref, o_ref):
        h1 = _sage_update(x_ref[...], acc_ref, cnt_ref, wa_ref[...], ba_ref[...])
        p = _dot(h1, w1_ref[...]) + b1_ref[...]
        q = _dot(p, w2_ref[...]) + b2_ref[...]
        m = jnp.max(q, axis=1, keepdims=True)
        lse = m + jnp.log(jnp.sum(jnp.exp(q - m), axis=1, keepdims=True))
        o_ref[...] = q - lse

    return pl.pallas_call(
        body,
        grid=(N // BB,),
        in_specs=[pl.BlockSpec((BB, H), lambda i: (i, 0)),
                  pl.BlockSpec((NC, BB, H), lambda i: (0, i, 0)),
                  pl.BlockSpec((NC, BB, H), lambda i: (0, i, 0)),
                  pl.BlockSpec((D + H, H), lambda i: (0, 0)),
                  pl.BlockSpec((1, H), lambda i: (0, 0)),
                  pl.BlockSpec((H, H), lambda i: (0, 0)),
                  pl.BlockSpec((1, H), lambda i: (0, 0)),
                  pl.BlockSpec((H, O), lambda i: (0, 0)),
                  pl.BlockSpec((1, O), lambda i: (0, 0))],
        out_specs=pl.BlockSpec((BB, O), lambda i: (i, 0)),
        out_shape=jax.ShapeDtypeStruct((N, O), jnp.float32),
    )(h0, acc, cnt, waT, ba, wp1T, bp1, wp2T, bp2)


def kernel(x, edge_index, W_lin0, b_lin0, W_agg0, b_agg0,
           W_lin1, b_lin1, W_agg1, b_agg1, Wp1, bp1, Wp2, bp2):
    npad = EPW - E // NW
    pad_src = (jnp.arange(npad, dtype=jnp.int32)[None, :] * 41
               + 97 * jnp.arange(NW, dtype=jnp.int32)[:, None]) % N
    pad_dst = jnp.broadcast_to(N + jnp.arange(npad, dtype=jnp.int32)[None, :],
                               (NW, npad))
    src_p = jnp.concatenate(
        [edge_index[0].reshape(NW, -1), pad_src], axis=1).reshape(EP)
    dst_p = jnp.concatenate(
        [edge_index[1].reshape(NW, -1), pad_dst], axis=1).reshape(EP)
    h = _tc_lin0(x, W_lin0.T, b_lin0.reshape(1, H))
    cnt, acc0 = _sc_aggregate(h, src_p, dst_p, with_counts=True)
    h0, h1l = _tc_update0(x, acc0, cnt, W_agg0.T, b_agg0.reshape(1, H),
                          W_lin1.T, b_lin1.reshape(1, H))
    acc1 = _sc_aggregate(h1l, src_p, dst_p)
    return _tc_final(h0, acc1, cnt, W_agg1.T, b_agg1.reshape(1, H),
                     Wp1.T, bp1.reshape(1, H), Wp2.T, bp2.reshape(1, O))


# speedup vs baseline: 9.6177x; 1.0577x over previous
import functools

import jax
import jax.numpy as jnp
from jax import lax
from jax.experimental import pallas as pl
from jax.experimental.pallas import tpu as pltpu
from jax.experimental.pallas import tpu_sc as plsc

N = 10000
E = 320000
D = 128
H = 128
O = 64

NC = 2
NS = 16
NW = NC * NS
CH = 128
NCH = 80
NPAIR = NCH // 2
EPW = CH * NCH
EP = NW * EPW
NP = 10240
RPT = NP // NS
ZB = 64

@functools.cache
def _mesh():
    return plsc.VectorSubcoreMesh(core_axis_name="c", subcore_axis_name="s")


def _sc_aggregate(h, src3, dst3, with_counts=False):
    out_t = jax.ShapeDtypeStruct((NC, NP, H), jnp.float32)

    @functools.partial(
        pl.kernel,
        out_type=[out_t, out_t] if with_counts else out_t,
        mesh=_mesh(),
        scratch_types=[
            [pltpu.VMEM((CH,), jnp.int32) for _ in range(2)],
            [pltpu.VMEM((CH,), jnp.int32) for _ in range(2)],
            [pltpu.VMEM((CH, H), jnp.float32) for _ in range(2)],
            pltpu.VMEM((ZB, H), jnp.float32),
            pltpu.VMEM_SHARED((NP, H), jnp.float32),
            [pltpu.SemaphoreType.DMA for _ in range(2)],
            [pltpu.SemaphoreType.DMA for _ in range(2)],
            [pltpu.SemaphoreType.DMA for _ in range(2)],
            [pltpu.SemaphoreType.DMA for _ in range(2)],
        ],
    )
    def k(h_hbm, src_hbm, dst_hbm, *rest):
        if with_counts:
            cnt_hbm, out_hbm = rest[0], rest[1]
            srcb, dstb, rows, zbuf, acc, isem, dsem, gsem, ssem = rest[2:]
        else:
            out_hbm = rest[0]
            srcb, dstb, rows, zbuf, acc, isem, dsem, gsem, ssem = rest[1:]
        c = lax.axis_index("c")
        s = lax.axis_index("s")
        base_e = (c * NS + s) * EPW

        def start_src(j, b):
            pltpu.async_copy(src_hbm.at[pl.ds(base_e + j * CH, CH)],
                             srcb[b], isem[b])

        def wait_src(b):
            pltpu.make_async_copy(src_hbm.at[pl.ds(base_e, CH)], srcb[b],
                                  isem[b]).wait()

        def start_dst(j, b):
            pltpu.async_copy(dst_hbm.at[pl.ds(base_e + j * CH, CH)],
                             dstb[b], dsem[b])

        def wait_dst(b):
            pltpu.make_async_copy(dst_hbm.at[pl.ds(base_e, CH)], dstb[b],
                                  dsem[b]).wait()

        start_src(0, 0)
        start_src(1, 1)

        @pl.loop(0, ZB)
        def _(r):
            @pl.loop(0, H // 16)
            def _(q):
                zbuf[r, pl.ds(q * 16, 16)] = jnp.zeros((16,), jnp.float32)

        @pl.loop(0, RPT // ZB)
        def _(i):
            pltpu.sync_copy(zbuf, acc.at[pl.ds(s * RPT + i * ZB, ZB)])

        plsc.subcore_barrier()

        if with_counts:
            @pl.loop(0, CH)
            def _(r):
                @pl.loop(0, H // 16)
                def _(q):
                    rows[1][r, pl.ds(q * 16, 16)] = jnp.ones((16,),
                                                             jnp.float32)

            def wait_ones_scat(b):
                pltpu.make_async_copy(rows[1], acc.at[dstb[b]],
                                      ssem[b]).wait()

            start_dst(0, 0)
            start_dst(1, 1)

            @pl.loop(0, NPAIR)
            def _(i):
                for b in range(2):
                    wait_dst(b)
                    pltpu.async_copy(rows[1], acc.at[dstb[b]], ssem[b],
                                     add=True)

                for b in range(2):
                    @pl.when(i < NPAIR - 1)
                    def _():
                        wait_ones_scat(b)
                        start_dst(2 * i + 2 + b, b)

            for b in range(2):
                wait_ones_scat(b)

            plsc.subcore_barrier()
            pltpu.sync_copy(acc.at[pl.ds(s * RPT, RPT)],
                            cnt_hbm.at[c, pl.ds(s * RPT, RPT)])

            @pl.loop(0, RPT // ZB)
            def _(i):
                pltpu.sync_copy(zbuf, acc.at[pl.ds(s * RPT + i * ZB, ZB)])

            plsc.subcore_barrier()

        def start_gather(b):
            pltpu.async_copy(h_hbm.at[srcb[b]], rows[b], gsem[b])

        def wait_gather(b):
            pltpu.make_async_copy(h_hbm.at[srcb[b]], rows[b], gsem[b]).wait()

        def start_scat(b):
            pltpu.async_copy(rows[b], acc.at[dstb[b]], ssem[b], add=True)

        def wait_scat(b):
            pltpu.make_async_copy(rows[b], acc.at[dstb[b]], ssem[b]).wait()

        @pl.loop(0, NPAIR)
        def _(i):
            wait_src(0)

            @pl.when(i > 0)
            def _():
                wait_scat(0)

            start_dst(2 * i, 0)
            start_gather(0)

            @pl.when(i > 0)
            def _():
                wait_gather(1)
                wait_dst(1)
                start_scat(1)
                start_src(2 * i + 1, 1)

            wait_src(1)

            @pl.when(i > 0)
            def _():
                wait_scat(1)

            start_dst(2 * i + 1, 1)
            start_gather(1)

            wait_gather(0)
            wait_dst(0)
            start_scat(0)

            @pl.when(i < NPAIR - 1)
            def _():
                start_src(2 * i + 2, 0)

        wait_gather(1)
        wait_dst(1)
        start_scat(1)
        wait_scat(0)
        wait_scat(1)
        plsc.subcore_barrier()
        pltpu.sync_copy(acc.at[pl.ds(s * RPT, RPT)],
                        out_hbm.at[c, pl.ds(s * RPT, RPT)])

    return k(h, src3, dst3)


BB = 2000


def _dot(a, b):
    return lax.dot_general(a, b, (((1,), (0,)), ((), ())),
                           precision=lax.Precision.HIGHEST,
                           preferred_element_type=jnp.float32)


def _tc_lin0(x, wT, b):
    def body(x_ref, w_ref, b_ref, o_ref):
        o_ref[...] = jnp.maximum(_dot(x_ref[...], w_ref[...]) + b_ref[...], 0.0)

    return pl.pallas_call(
        body,
        grid=(N // BB,),
        in_specs=[pl.BlockSpec((BB, D), lambda i: (i, 0)),
                  pl.BlockSpec((D, H), lambda i: (0, 0)),
                  pl.BlockSpec((1, H), lambda i: (0, 0))],
        out_specs=pl.BlockSpec((BB, H), lambda i: (i, 0)),
        out_shape=jax.ShapeDtypeStruct((N, H), jnp.float32),
    )(x, wT, b)


def _sage_update(x_blk, acc_ref, cnt_ref, waT, ba):
    cnt = cnt_ref[0, :, 0:1] + cnt_ref[1, :, 0:1]
    agg = (acc_ref[0] + acc_ref[1]) / jnp.maximum(cnt, 1.0)
    cat = jnp.concatenate([x_blk, agg], axis=1)
    o = jnp.maximum(_dot(cat, waT) + ba, 0.0)
    nrm = jnp.maximum(jnp.sqrt(jnp.sum(o * o, axis=1, keepdims=True)), 1e-12)
    return o / nrm


def _tc_update0(x, acc, cnt, waT, ba, wlT, bl):
    def body(x_ref, acc_ref, cnt_ref, wa_ref, ba_ref, wl_ref, bl_ref,
             h0_ref, h1_ref):
        h0 = _sage_update(x_ref[...], acc_ref, cnt_ref, wa_ref[...], ba_ref[...])
        h0_ref[...] = h0
        h1_ref[...] = jnp.maximum(_dot(h0, wl_ref[...]) + bl_ref[...], 0.0)

    return pl.pallas_call(
        body,
        grid=(N // BB,),
        in_specs=[pl.BlockSpec((BB, H), lambda i: (i, 0)),
                  pl.BlockSpec((NC, BB, H), lambda i: (0, i, 0)),
                  pl.BlockSpec((NC, BB, H), lambda i: (0, i, 0)),
                  pl.BlockSpec((D + H, H), lambda i: (0, 0)),
                  pl.BlockSpec((1, H), lambda i: (0, 0)),
                  pl.BlockSpec((H, H), lambda i: (0, 0)),
                  pl.BlockSpec((1, H), lambda i: (0, 0))],
        out_specs=[pl.BlockSpec((BB, H), lambda i: (i, 0)),
                   pl.BlockSpec((BB, H), lambda i: (i, 0))],
        out_shape=[jax.ShapeDtypeStruct((N, H), jnp.float32),
                   jax.ShapeDtypeStruct((N, H), jnp.float32)],
    )(x, acc, cnt, waT, ba, wlT, bl)


def _tc_final(h0, acc, cnt, waT, ba, wp1T, bp1, wp2T, bp2):
    def body(x_ref, acc_ref, cnt_ref, wa_ref, ba_ref, w1_ref, b1_ref,
             w2_ref, b2_ref, o_ref):
        h1 = _sage_update(x_ref[...], acc_ref, cnt_ref, wa_ref[...], ba_ref[...])
        p = _dot(h1, w1_ref[...]) + b1_ref[...]
        q = _dot(p, w2_ref[...]) + b2_ref[...]
        m = jnp.max(q, axis=1, keepdims=True)
        lse = m + jnp.log(jnp.sum(jnp.exp(q - m), axis=1, keepdims=True))
        o_ref[...] = q - lse

    return pl.pallas_call(
        body,
        grid=(N // BB,),
        in_specs=[pl.BlockSpec((BB, H), lambda i: (i, 0)),
                  pl.BlockSpec((NC, BB, H), lambda i: (0, i, 0)),
                  pl.BlockSpec((NC, BB, H), lambda i: (0, i, 0)),
                  pl.BlockSpec((D + H, H), lambda i: (0, 0)),
                  pl.BlockSpec((1, H), lambda i: (0, 0)),
                  pl.BlockSpec((H, H), lambda i: (0, 0)),
                  pl.BlockSpec((1, H), lambda i: (0, 0)),
                  pl.BlockSpec((H, O), lambda i: (0, 0)),
                  pl.BlockSpec((1, O), lambda i: (0, 0))],
        out_specs=pl.BlockSpec((BB, O), lambda i: (i, 0)),
        out_shape=jax.ShapeDtypeStruct((N, O), jnp.float32),
    )(h0, acc, cnt, waT, ba, wp1T, bp1, wp2T, bp2)


def kernel(x, edge_index, W_lin0, b_lin0, W_agg0, b_agg0,
           W_lin1, b_lin1, W_agg1, b_agg1, Wp1, bp1, Wp2, bp2):
    npad = EPW - E // NW
    pad_src = (jnp.arange(npad, dtype=jnp.int32)[None, :] * 41
               + 97 * jnp.arange(NW, dtype=jnp.int32)[:, None]) % N
    pad_dst = jnp.broadcast_to(N + jnp.arange(npad, dtype=jnp.int32)[None, :],
                               (NW, npad))
    src_p = jnp.concatenate(
        [edge_index[0].reshape(NW, -1), pad_src], axis=1).reshape(EP)
    dst_p = jnp.concatenate(
        [edge_index[1].reshape(NW, -1), pad_dst], axis=1).reshape(EP)
    h = _tc_lin0(x, W_lin0.T, b_lin0.reshape(1, H))
    cnt, acc0 = _sc_aggregate(h, src_p, dst_p, with_counts=True)
    h0, h1l = _tc_update0(x, acc0, cnt, W_agg0.T, b_agg0.reshape(1, H),
                          W_lin1.T, b_lin1.reshape(1, H))
    acc1 = _sc_aggregate(h1l, src_p, dst_p)
    return _tc_final(h0, acc1, cnt, W_agg1.T, b_agg1.reshape(1, H),
                     Wp1.T, bp1.reshape(1, H), Wp2.T, bp2.reshape(1, O))


# speedup vs baseline: 9.8234x; 1.0214x over previous
import functools

import jax
import jax.numpy as jnp
from jax import lax
from jax.experimental import pallas as pl
from jax.experimental.pallas import tpu as pltpu
from jax.experimental.pallas import tpu_sc as plsc

N = 10000
E = 320000
D = 128
H = 128
O = 64

NC = 2
NS = 16
NW = NC * NS
CH = 128
NCH = 80
NPAIR = NCH // 2
EPW = CH * NCH
EP = NW * EPW
NP = 10240
RPT = NP // NS
ZB = 64

@functools.cache
def _mesh():
    return plsc.VectorSubcoreMesh(core_axis_name="c", subcore_axis_name="s")


def _sc_aggregate(h, e2, with_counts=False):
    out_t = jax.ShapeDtypeStruct((NC, NP, H), jnp.float32)
    NT = NCH // 4

    @functools.partial(
        pl.kernel,
        out_type=[out_t, out_t] if with_counts else out_t,
        mesh=_mesh(),
        scratch_types=[
            [pltpu.VMEM((2, CH), jnp.int32) for _ in range(4)],
            [pltpu.VMEM((CH, H), jnp.float32) for _ in range(2)],
            pltpu.VMEM((ZB, H), jnp.float32),
            pltpu.VMEM_SHARED((NP, H), jnp.float32),
            [pltpu.SemaphoreType.DMA for _ in range(4)],
            [pltpu.SemaphoreType.DMA for _ in range(2)],
            [pltpu.SemaphoreType.DMA for _ in range(2)],
        ],
    )
    def k(h_hbm, e2_hbm, *rest):
        if with_counts:
            cnt_hbm, out_hbm = rest[0], rest[1]
            ibuf, rows, zbuf, acc, isem, gsem, ssem = rest[2:]
        else:
            out_hbm = rest[0]
            ibuf, rows, zbuf, acc, isem, gsem, ssem = rest[1:]
        c = lax.axis_index("c")
        s = lax.axis_index("s")
        base_c = (c * NS + s) * NCH

        def start_idx(j, u):
            pltpu.async_copy(e2_hbm.at[base_c + j], ibuf[u], isem[u])

        def wait_idx(u):
            pltpu.make_async_copy(e2_hbm.at[base_c], ibuf[u], isem[u]).wait()

        def start_gather(u, p):
            pltpu.async_copy(h_hbm.at[ibuf[u].at[0]], rows[p], gsem[p])

        def wait_gather(u, p):
            pltpu.make_async_copy(h_hbm.at[ibuf[u].at[0]], rows[p],
                                  gsem[p]).wait()

        def start_scat(u, p, src_rows):
            pltpu.async_copy(src_rows, acc.at[ibuf[u].at[1]], ssem[p],
                             add=True)

        def wait_scat(u, p, src_rows):
            pltpu.make_async_copy(src_rows, acc.at[ibuf[u].at[1]],
                                  ssem[p]).wait()

        def refill(t, cq):
            u2 = (cq + 2) % 4
            if cq < 2:
                start_idx(4 * t + cq + 2, u2)
            else:
                @pl.when(t < NT - 1)
                def _():
                    start_idx(4 * t + cq + 2, u2)

        for u in range(4):
            start_idx(u, u)

        @pl.loop(0, ZB)
        def _(r):
            @pl.loop(0, H // 16)
            def _(q):
                zbuf[r, pl.ds(q * 16, 16)] = jnp.zeros((16,), jnp.float32)

        @pl.loop(0, RPT // ZB)
        def _(i):
            pltpu.sync_copy(zbuf, acc.at[pl.ds(s * RPT + i * ZB, ZB)])

        plsc.subcore_barrier()

        if with_counts:
            @pl.loop(0, CH)
            def _(r):
                @pl.loop(0, H // 16)
                def _(q):
                    rows[1][r, pl.ds(q * 16, 16)] = jnp.ones((16,),
                                                             jnp.float32)

            @pl.loop(0, NT)
            def _(t):
                for cq in range(4):
                    u, p, u2 = cq, cq % 2, (cq + 2) % 4
                    wait_idx(u)
                    if cq < 2:
                        @pl.when(t > 0)
                        def _():
                            wait_scat(u2, p, rows[1])
                            refill(t, cq)
                    else:
                        wait_scat(u2, p, rows[1])
                        refill(t, cq)
                    start_scat(u, p, rows[1])

            wait_scat(2, 0, rows[1])
            wait_scat(3, 1, rows[1])
            plsc.subcore_barrier()
            pltpu.sync_copy(acc.at[pl.ds(s * RPT, RPT)],
                            cnt_hbm.at[c, pl.ds(s * RPT, RPT)])

            @pl.loop(0, RPT // ZB)
            def _(i):
                pltpu.sync_copy(zbuf, acc.at[pl.ds(s * RPT + i * ZB, ZB)])

            plsc.subcore_barrier()

            for u in range(4):
                start_idx(u, u)

        @pl.loop(0, NT)
        def _(t):
            for cq in range(4):
                u, p, u2, um, q = cq, cq % 2, (cq + 2) % 4, (cq - 1) % 4, \
                    1 - cq % 2
                wait_idx(u)
                if cq < 2:
                    @pl.when(t > 0)
                    def _():
                        wait_scat(u2, p, rows[p])
                        refill(t, cq)
                else:
                    wait_scat(u2, p, rows[p])
                    refill(t, cq)
                start_gather(u, p)
                if cq == 0:
                    @pl.when(t > 0)
                    def _():
                        wait_gather(um, q)
                        start_scat(um, q, rows[q])
                else:
                    wait_gather(um, q)
                    start_scat(um, q, rows[q])

        wait_gather(3, 1)
        start_scat(3, 1, rows[1])
        wait_scat(2, 0, rows[0])
        wait_scat(3, 1, rows[1])
        plsc.subcore_barrier()
        pltpu.sync_copy(acc.at[pl.ds(s * RPT, RPT)],
                        out_hbm.at[c, pl.ds(s * RPT, RPT)])

    return k(h, e2)


BB = 2000


def _dot(a, b):
    return lax.dot_general(a, b, (((1,), (0,)), ((), ())),
                           precision=lax.Precision.HIGHEST,
                           preferred_element_type=jnp.float32)


def _tc_lin0(x, wT, b):
    def body(x_ref, w_ref, b_ref, o_ref):
        o_ref[...] = jnp.maximum(_dot(x_ref[...], w_ref[...]) + b_ref[...], 0.0)

    return pl.pallas_call(
        body,
        grid=(N // BB,),
        in_specs=[pl.BlockSpec((BB, D), lambda i: (i, 0)),
                  pl.BlockSpec((D, H), lambda i: (0, 0)),
                  pl.BlockSpec((1, H), lambda i: (0, 0))],
        out_specs=pl.BlockSpec((BB, H), lambda i: (i, 0)),
        out_shape=jax.ShapeDtypeStruct((N, H), jnp.float32),
    )(x, wT, b)


def _sage_update(x_blk, acc_ref, cnt_ref, waT, ba):
    cnt = cnt_ref[0, :, 0:1] + cnt_ref[1, :, 0:1]
    agg = (acc_ref[0] + acc_ref[1]) / jnp.maximum(cnt, 1.0)
    cat = jnp.concatenate([x_blk, agg], axis=1)
    o = jnp.maximum(_dot(cat, waT) + ba, 0.0)
    nrm = jnp.maximum(jnp.sqrt(jnp.sum(o * o, axis=1, keepdims=True)), 1e-12)
    return o / nrm


def _tc_update0(x, acc, cnt, waT, ba, wlT, bl):
    def body(x_ref, acc_ref, cnt_ref, wa_ref, ba_ref, wl_ref, bl_ref,
             h0_ref, h1_ref):
        h0 = _sage_update(x_ref[...], acc_ref, cnt_ref, wa_ref[...], ba_ref[...])
        h0_ref[...] = h0
        h1_ref[...] = jnp.maximum(_dot(h0, wl_ref[...]) + bl_ref[...], 0.0)

    return pl.pallas_call(
        body,
        grid=(N // BB,),
        in_specs=[pl.BlockSpec((BB, H), lambda i: (i, 0)),
                  pl.BlockSpec((NC, BB, H), lambda i: (0, i, 0)),
                  pl.BlockSpec((NC, BB, H), lambda i: (0, i, 0)),
                  pl.BlockSpec((D + H, H), lambda i: (0, 0)),
                  pl.BlockSpec((1, H), lambda i: (0, 0)),
                  pl.BlockSpec((H, H), lambda i: (0, 0)),
                  pl.BlockSpec((1, H), lambda i: (0, 0))],
        out_specs=[pl.BlockSpec((BB, H), lambda i: (i, 0)),
                   pl.BlockSpec((BB, H), lambda i: (i, 0))],
        out_shape=[jax.ShapeDtypeStruct((N, H), jnp.float32),
                   jax.ShapeDtypeStruct((N, H), jnp.float32)],
    )(x, acc, cnt, waT, ba, wlT, bl)


def _tc_final(h0, acc, cnt, waT, ba, wp1T, bp1, wp2T, bp2):
    def body(x_ref, acc_ref, cnt_ref, wa_ref, ba_ref, w1_ref, b1_ref,
             w2_ref, b2_ref, o_ref):
        h1 = _sage_update(x_ref[...], acc_ref, cnt_ref, wa_ref[...], ba_ref[...])
        p = _dot(h1, w1_ref[...]) + b1_ref[...]
        q = _dot(p, w2_ref[...]) + b2_ref[...]
        m = jnp.max(q, axis=1, keepdims=True)
        lse = m + jnp.log(jnp.sum(jnp.exp(q - m), axis=1, keepdims=True))
        o_ref[...] = q - lse

    return pl.pallas_call(
        body,
        grid=(N // BB,),
        in_specs=[pl.BlockSpec((BB, H), lambda i: (i, 0)),
                  pl.BlockSpec((NC, BB, H), lambda i: (0, i, 0)),
                  pl.BlockSpec((NC, BB, H), lambda i: (0, i, 0)),
                  pl.BlockSpec((D + H, H), lambda i: (0, 0)),
                  pl.BlockSpec((1, H), lambda i: (0, 0)),
                  pl.BlockSpec((H, H), lambda i: (0, 0)),
                  pl.BlockSpec((1, H), lambda i: (0, 0)),
                  pl.BlockSpec((H, O), lambda i: (0, 0)),
                  pl.BlockSpec((1, O), lambda i: (0, 0))],
        out_specs=pl.BlockSpec((BB, O), lambda i: (i, 0)),
        out_shape=jax.ShapeDtypeStruct((N, O), jnp.float32),
    )(h0, acc, cnt, waT, ba, wp1T, bp1, wp2T, bp2)


def kernel(x, edge_index, W_lin0, b_lin0, W_agg0, b_agg0,
           W_lin1, b_lin1, W_agg1, b_agg1, Wp1, bp1, Wp2, bp2):
    npad = EPW - E // NW
    pad_src = (jnp.arange(npad, dtype=jnp.int32)[None, :] * 41
               + 97 * jnp.arange(NW, dtype=jnp.int32)[:, None]) % N
    pad_dst = jnp.broadcast_to(N + jnp.arange(npad, dtype=jnp.int32)[None, :],
                               (NW, npad))
    src_p = jnp.concatenate(
        [edge_index[0].reshape(NW, -1), pad_src], axis=1).reshape(EP)
    dst_p = jnp.concatenate(
        [edge_index[1].reshape(NW, -1), pad_dst], axis=1).reshape(EP)
    h = _tc_lin0(x, W_lin0.T, b_lin0.reshape(1, H))
    e2 = jnp.stack([src_p.reshape(NW * NCH, CH),
                    dst_p.reshape(NW * NCH, CH)], axis=1)
    cnt, acc0 = _sc_aggregate(h, e2, with_counts=True)
    h0, h1l = _tc_update0(x, acc0, cnt, W_agg0.T, b_agg0.reshape(1, H),
                          W_lin1.T, b_lin1.reshape(1, H))
    acc1 = _sc_aggregate(h1l, e2)
    return _tc_final(h0, acc1, cnt, W_agg1.T, b_agg1.reshape(1, H),
                     Wp1.T, bp1.reshape(1, H), Wp2.T, bp2.reshape(1, O))
